# baseline (speedup 1.0000x reference)
import functools

import jax
import jax.numpy as jnp
from jax import lax
from jax.experimental import pallas as pl
from jax.experimental.pallas import tpu as pltpu
from jax.experimental.pallas import tpu_sc as plsc

_NUM_EXPERTS = 16
_HIDDEN = 1024
_INTER = 2048
_JITTER = 0.01
_IC = 1024
_NC = _INTER // _IC
_LANES = 16
_NWORKERS = 32
_HCHUNKS = _HIDDEN // _LANES


def _sparsemixer_row(lv):
    neg_inf = jnp.float32(-jnp.inf)
    lane = lax.iota(jnp.int32, _LANES)
    t1 = lax.reduce_max(lv, (0,))
    oh1 = lane == plsc.all_reduce_ffs(lv == t1)
    factor1 = jnp.maximum(jnp.abs(lv), t1)
    mask1 = ((t1 - lv) / factor1) > (2.0 * _JITTER)
    p1 = jnp.exp(jnp.where(mask1, neg_inf, lv) - t1)
    den1 = jnp.broadcast_to(lax.reduce_sum(p1, (0,)), (_LANES,))
    row1 = jnp.where(oh1, p1, 0.0) / den1
    s2 = jnp.where(oh1, neg_inf, lv)
    t2 = lax.reduce_max(s2, (0,))
    oh2 = lane == plsc.all_reduce_ffs(s2 == t2)
    factor2 = jnp.maximum(jnp.abs(lv), t2)
    mask2 = ((t2 - s2) / factor2) > (2.0 * _JITTER)
    p2 = jnp.exp(jnp.where(mask2, neg_inf, s2) - t2)
    den2 = jnp.broadcast_to(lax.reduce_sum(p2, (0,)), (_LANES,))
    row2 = jnp.where(oh2, p2, 0.0) / den2
    return row1 + row2


def _bf16_round(v):
    u = plsc.bitcast(v, jnp.uint32)
    r = (u + jnp.uint32(0x7FFF) + ((u >> 16) & jnp.uint32(1))) & jnp.uint32(0xFFFF0000)
    return plsc.bitcast(r, jnp.float32)


def _router_body(x_hbm, gw_hbm, out_hbm, xv, gv, wv):
    wid = lax.axis_index("s") * 2 + lax.axis_index("c")
    tpw = 4
    base = wid * tpw
    pltpu.sync_copy(x_hbm.at[pl.ds(base, tpw)], xv)
    pltpu.sync_copy(gw_hbm, gv)

    for t0 in range(0, tpw, 2):
        def step(k, accs):
            o = k * _LANES
            xc0 = _bf16_round(xv[t0, pl.ds(o, _LANES)])
            xc1 = _bf16_round(xv[t0 + 1, pl.ds(o, _LANES)])
            new = []
            for e in range(_NUM_EXPERTS):
                gc = _bf16_round(gv[e, pl.ds(o, _LANES)])
                new.append(accs[e] + xc0 * gc)
                new.append(accs[_NUM_EXPERTS + e] + xc1 * gc)
            return tuple(new[0::2]) + tuple(new[1::2])

        accs = lax.fori_loop(
            0, _HCHUNKS, step,
            tuple(jnp.zeros((_LANES,), jnp.float32) for _ in range(2 * _NUM_EXPERTS)))
        for dt in range(2):
            parts = [lax.reduce_sum(accs[dt * _NUM_EXPERTS + e], (0,))
                     for e in range(_NUM_EXPERTS)]
            lane = lax.iota(jnp.int32, _LANES)
            lv = jnp.zeros((_LANES,), jnp.float32)
            for e in range(_NUM_EXPERTS):
                lv = jnp.where(lane == e, parts[e], lv)
            wv[t0 + dt, :] = _sparsemixer_row(lv)

    pltpu.sync_copy(wv, out_hbm.at[pl.ds(base, tpw)])


def _sc_router(x, gate_w):
    tokens = x.shape[0]
    mesh = plsc.VectorSubcoreMesh(core_axis_name="c", subcore_axis_name="s")
    tpw = tokens // _NWORKERS
    run = pl.kernel(
        _router_body, mesh=mesh,
        out_type=jax.ShapeDtypeStruct((tokens, _NUM_EXPERTS), jnp.float32),
        scratch_types=[
            pltpu.VMEM((tpw, _HIDDEN), jnp.float32),
            pltpu.VMEM((_NUM_EXPERTS, _HIDDEN), jnp.float32),
            pltpu.VMEM((tpw, _NUM_EXPERTS), jnp.float32),
        ],
        compiler_params=pltpu.CompilerParams(needs_layout_passes=False),
    )
    return run(x, gate_w)


def _moe_body(x_ref, w1_ref, w3_ref, w2_ref, out_ref):
    c = pl.program_id(1)

    xb = x_ref[...].astype(jnp.bfloat16)
    w1b = w1_ref[0].astype(jnp.bfloat16)
    w3b = w3_ref[0].astype(jnp.bfloat16)
    a = jax.lax.dot_general(xb, w1b, (((1,), (1,)), ((), ())),
                            preferred_element_type=jnp.float32)
    g = jax.lax.dot_general(xb, w3b, (((1,), (1,)), ((), ())),
                            preferred_element_type=jnp.float32)
    h = (a * jax.nn.sigmoid(a)) * g
    w2b = w2_ref[0].astype(jnp.bfloat16)
    contrib = jax.lax.dot_general(h.astype(jnp.bfloat16), w2b,
                                  (((1,), (1,)), ((), ())),
                                  preferred_element_type=jnp.float32)

    @pl.when(c == 0)
    def _():
        out_ref[0] = contrib

    @pl.when(c != 0)
    def _():
        out_ref[0] += contrib


def _combine_body(o_ref, wts_ref, out_ref):
    e = pl.program_id(0)
    tokens = out_ref.shape[0]
    eidx = jax.lax.broadcasted_iota(jnp.int32, (tokens, _NUM_EXPERTS), 1)
    wcol = jnp.sum(jnp.where(eidx == e, wts_ref[...], 0.0), axis=1,
                   keepdims=True)
    upd = o_ref[0] * wcol

    @pl.when(e == 0)
    def _():
        out_ref[...] = upd

    @pl.when(e != 0)
    def _():
        out_ref[...] += upd


def kernel(hidden_states, gate_w, w1, w2, w3):
    b, s, hdim = hidden_states.shape
    tokens = b * s
    x = hidden_states.reshape(tokens, hdim)
    expert_out = pl.pallas_call(
        _moe_body,
        grid=(_NUM_EXPERTS, _NC),
        in_specs=[
            pl.BlockSpec((tokens, _HIDDEN), lambda e, c: (0, 0)),
            pl.BlockSpec((1, _IC, _HIDDEN), lambda e, c: (e, c, 0)),
            pl.BlockSpec((1, _IC, _HIDDEN), lambda e, c: (e, c, 0)),
            pl.BlockSpec((1, _HIDDEN, _IC), lambda e, c: (e, 0, c)),
        ],
        out_specs=pl.BlockSpec((1, tokens, _HIDDEN), lambda e, c: (e, 0, 0)),
        out_shape=jax.ShapeDtypeStruct((_NUM_EXPERTS, tokens, _HIDDEN),
                                       jnp.float32),
        compiler_params=pltpu.CompilerParams(
            dimension_semantics=("arbitrary", "arbitrary")),
    )(x, w1, w3, w2)
    wts = _sc_router(x, gate_w)
    out = pl.pallas_call(
        _combine_body,
        grid=(_NUM_EXPERTS,),
        in_specs=[
            pl.BlockSpec((1, tokens, _HIDDEN), lambda e: (e, 0, 0)),
            pl.BlockSpec((tokens, _NUM_EXPERTS), lambda e: (0, 0)),
        ],
        out_specs=pl.BlockSpec((tokens, _HIDDEN), lambda e: (0, 0)),
        out_shape=jax.ShapeDtypeStruct((tokens, _HIDDEN), jnp.float32),
        compiler_params=pltpu.CompilerParams(
            dimension_semantics=("arbitrary",)),
    )(expert_out, wts)
    return out.reshape(b, s, hdim)


# speedup vs baseline: 1.0562x; 1.0562x over previous
import jax
import jax.numpy as jnp
from jax import lax
from jax.experimental import pallas as pl
from jax.experimental.pallas import tpu as pltpu
from jax.experimental.pallas import tpu_sc as plsc

_NUM_EXPERTS = 16
_HIDDEN = 1024
_INTER = 2048
_JITTER = 0.01
_IC = 1024
_NC = _INTER // _IC
_LANES = 16
_NWORKERS = 32
_HCHUNKS = _HIDDEN // _LANES


def _sparsemixer_row(lv):
    neg_inf = jnp.float32(-jnp.inf)
    lane = lax.iota(jnp.int32, _LANES)
    t1 = lax.reduce_max(lv, (0,))
    oh1 = lane == plsc.all_reduce_ffs(lv == t1)
    factor1 = jnp.maximum(jnp.abs(lv), t1)
    mask1 = ((t1 - lv) / factor1) > (2.0 * _JITTER)
    p1 = jnp.exp(jnp.where(mask1, neg_inf, lv) - t1)
    den1 = jnp.broadcast_to(lax.reduce_sum(p1, (0,)), (_LANES,))
    row1 = jnp.where(oh1, p1, 0.0) / den1
    s2 = jnp.where(oh1, neg_inf, lv)
    t2 = lax.reduce_max(s2, (0,))
    oh2 = lane == plsc.all_reduce_ffs(s2 == t2)
    factor2 = jnp.maximum(jnp.abs(lv), t2)
    mask2 = ((t2 - s2) / factor2) > (2.0 * _JITTER)
    p2 = jnp.exp(jnp.where(mask2, neg_inf, s2) - t2)
    den2 = jnp.broadcast_to(lax.reduce_sum(p2, (0,)), (_LANES,))
    row2 = jnp.where(oh2, p2, 0.0) / den2
    return row1 + row2


def _router_body(x_hbm, gw_hbm, out_hbm, xv, gv, wv):
    wid = lax.axis_index("s") * 2 + lax.axis_index("c")
    tpw = 4
    base = wid * tpw
    pltpu.sync_copy(x_hbm.at[pl.ds(base, tpw)], xv)
    pltpu.sync_copy(gw_hbm, gv)

    for t0 in range(0, tpw, 2):
        def step(k, accs):
            o = k * _LANES
            xc0 = xv[t0, pl.ds(o, _LANES)]
            xc1 = xv[t0 + 1, pl.ds(o, _LANES)]
            new = []
            for e in range(_NUM_EXPERTS):
                gc = gv[e, pl.ds(o, _LANES)]
                new.append(accs[e] + xc0 * gc)
                new.append(accs[_NUM_EXPERTS + e] + xc1 * gc)
            return tuple(new[0::2]) + tuple(new[1::2])

        accs = lax.fori_loop(
            0, _HCHUNKS, step,
            tuple(jnp.zeros((_LANES,), jnp.float32) for _ in range(2 * _NUM_EXPERTS)))
        for dt in range(2):
            parts = [lax.reduce_sum(accs[dt * _NUM_EXPERTS + e], (0,))
                     for e in range(_NUM_EXPERTS)]
            lane = lax.iota(jnp.int32, _LANES)
            lv = jnp.zeros((_LANES,), jnp.float32)
            for e in range(_NUM_EXPERTS):
                lv = jnp.where(lane == e, parts[e], lv)
            wv[t0 + dt, :] = _sparsemixer_row(lv)

    pltpu.sync_copy(wv, out_hbm.at[pl.ds(base, tpw)])


def _sc_router(x, gate_w):
    tokens = x.shape[0]
    mesh = plsc.VectorSubcoreMesh(core_axis_name="c", subcore_axis_name="s")
    tpw = tokens // _NWORKERS
    run = pl.kernel(
        _router_body, mesh=mesh,
        out_type=jax.ShapeDtypeStruct((tokens, _NUM_EXPERTS), jnp.float32),
        scratch_types=[
            pltpu.VMEM((tpw, _HIDDEN), jnp.float32),
            pltpu.VMEM((_NUM_EXPERTS, _HIDDEN), jnp.float32),
            pltpu.VMEM((tpw, _NUM_EXPERTS), jnp.float32),
        ],
        compiler_params=pltpu.CompilerParams(needs_layout_passes=False),
    )
    return run(x, gate_w)


def _moe_body(x_ref, wts_ref, w1_ref, w3_ref, w2_ref, out_ref):
    e = pl.program_id(0)
    c = pl.program_id(1)
    first = (e == 0) & (c == 0)

    xb = x_ref[...].astype(jnp.bfloat16)
    w1b = w1_ref[0].astype(jnp.bfloat16)
    w3b = w3_ref[0].astype(jnp.bfloat16)
    a = jax.lax.dot_general(xb, w1b, (((1,), (1,)), ((), ())),
                            preferred_element_type=jnp.float32)
    g = jax.lax.dot_general(xb, w3b, (((1,), (1,)), ((), ())),
                            preferred_element_type=jnp.float32)
    h = (a * jax.nn.sigmoid(a)) * g
    w2b = w2_ref[0].astype(jnp.bfloat16)
    contrib = jax.lax.dot_general(h.astype(jnp.bfloat16), w2b,
                                  (((1,), (1,)), ((), ())),
                                  preferred_element_type=jnp.float32)
    tokens = contrib.shape[0]
    eidx = jax.lax.broadcasted_iota(jnp.int32, (tokens, _NUM_EXPERTS), 1)
    wcol = jnp.sum(jnp.where(eidx == e, wts_ref[...], 0.0), axis=1,
                   keepdims=True)
    upd = contrib * wcol

    @pl.when(first)
    def _():
        out_ref[...] = upd

    @pl.when(jnp.logical_not(first))
    def _():
        out_ref[...] += upd


def kernel(hidden_states, gate_w, w1, w2, w3):
    b, s, hdim = hidden_states.shape
    tokens = b * s
    x = hidden_states.reshape(tokens, hdim)
    x_r = x.astype(jnp.bfloat16).astype(jnp.float32)
    gw_r = gate_w.astype(jnp.bfloat16).astype(jnp.float32)
    wts = _sc_router(x_r, gw_r)
    out = pl.pallas_call(
        _moe_body,
        grid=(_NUM_EXPERTS, _NC),
        in_specs=[
            pl.BlockSpec((tokens, _HIDDEN), lambda e, c: (0, 0)),
            pl.BlockSpec((tokens, _NUM_EXPERTS), lambda e, c: (0, 0)),
            pl.BlockSpec((1, _IC, _HIDDEN), lambda e, c: (e, c, 0)),
            pl.BlockSpec((1, _IC, _HIDDEN), lambda e, c: (e, c, 0)),
            pl.BlockSpec((1, _HIDDEN, _IC), lambda e, c: (e, 0, c)),
        ],
        out_specs=pl.BlockSpec((tokens, _HIDDEN), lambda e, c: (0, 0)),
        out_shape=jax.ShapeDtypeStruct((tokens, _HIDDEN), jnp.float32),
        compiler_params=pltpu.CompilerParams(
            dimension_semantics=("arbitrary", "arbitrary")),
    )(x, wts, w1, w3, w2)
    return out.reshape(b, s, hdim)


# speedup vs baseline: 1.0562x; 1.0001x over previous
import jax
import jax.numpy as jnp
from jax import lax
from jax.experimental import pallas as pl
from jax.experimental.pallas import tpu as pltpu
from jax.experimental.pallas import tpu_sc as plsc

_NUM_EXPERTS = 16
_HIDDEN = 1024
_INTER = 2048
_JITTER = 0.01
_IC = 1024
_NC = _INTER // _IC
_LANES = 16
_NWORKERS = 32
_HCHUNKS = _HIDDEN // _LANES


def _sparsemixer_row(lv):
    neg_inf = jnp.float32(-jnp.inf)
    lane = lax.iota(jnp.int32, _LANES)
    t1 = lax.reduce_max(lv, (0,))
    oh1 = lane == plsc.all_reduce_ffs(lv == t1)
    factor1 = jnp.maximum(jnp.abs(lv), t1)
    mask1 = ((t1 - lv) / factor1) > (2.0 * _JITTER)
    p1 = jnp.exp(jnp.where(mask1, neg_inf, lv) - t1)
    den1 = jnp.broadcast_to(lax.reduce_sum(p1, (0,)), (_LANES,))
    row1 = jnp.where(oh1, p1, 0.0) / den1
    s2 = jnp.where(oh1, neg_inf, lv)
    t2 = lax.reduce_max(s2, (0,))
    oh2 = lane == plsc.all_reduce_ffs(s2 == t2)
    factor2 = jnp.maximum(jnp.abs(lv), t2)
    mask2 = ((t2 - s2) / factor2) > (2.0 * _JITTER)
    p2 = jnp.exp(jnp.where(mask2, neg_inf, s2) - t2)
    den2 = jnp.broadcast_to(lax.reduce_sum(p2, (0,)), (_LANES,))
    row2 = jnp.where(oh2, p2, 0.0) / den2
    return row1 + row2


def _router_body(x_hbm, gw_hbm, out_hbm, xv, gv, wv):
    wid = lax.axis_index("s") * 2 + lax.axis_index("c")
    tpw = 4
    base = wid * tpw
    pltpu.sync_copy(x_hbm.at[pl.ds(base, tpw)], xv)
    pltpu.sync_copy(gw_hbm, gv)

    for t0 in range(0, tpw, 2):
        def step(k, accs):
            o = k * _LANES
            xc0 = xv[t0, pl.ds(o, _LANES)]
            xc1 = xv[t0 + 1, pl.ds(o, _LANES)]
            new = []
            for e in range(_NUM_EXPERTS):
                gc = gv[e, pl.ds(o, _LANES)]
                new.append(accs[e] + xc0 * gc)
                new.append(accs[_NUM_EXPERTS + e] + xc1 * gc)
            return tuple(new[0::2]) + tuple(new[1::2])

        accs = lax.fori_loop(
            0, _HCHUNKS, step,
            tuple(jnp.zeros((_LANES,), jnp.float32) for _ in range(2 * _NUM_EXPERTS)))
        for dt in range(2):
            parts = [lax.reduce_sum(accs[dt * _NUM_EXPERTS + e], (0,))
                     for e in range(_NUM_EXPERTS)]
            lane = lax.iota(jnp.int32, _LANES)
            lv = jnp.zeros((_LANES,), jnp.float32)
            for e in range(_NUM_EXPERTS):
                lv = jnp.where(lane == e, parts[e], lv)
            wv[t0 + dt, :] = _sparsemixer_row(lv)

    pltpu.sync_copy(wv, out_hbm.at[pl.ds(base, tpw)])


def _sc_router(x, gate_w):
    tokens = x.shape[0]
    mesh = plsc.VectorSubcoreMesh(core_axis_name="c", subcore_axis_name="s")
    tpw = tokens // _NWORKERS
    run = pl.kernel(
        _router_body, mesh=mesh,
        out_type=jax.ShapeDtypeStruct((tokens, _NUM_EXPERTS), jnp.float32),
        scratch_types=[
            pltpu.VMEM((tpw, _HIDDEN), jnp.float32),
            pltpu.VMEM((_NUM_EXPERTS, _HIDDEN), jnp.float32),
            pltpu.VMEM((tpw, _NUM_EXPERTS), jnp.float32),
        ],
        compiler_params=pltpu.CompilerParams(needs_layout_passes=False),
    )
    return run(x, gate_w)


def _moe_body(x_ref, wts_ref, w1_ref, w3_ref, w2_ref, out_ref):
    e = pl.program_id(0)
    c = pl.program_id(1)
    first = (e == 0) & (c == 0)

    xb = x_ref[...].astype(jnp.bfloat16)
    w1b = w1_ref[0].astype(jnp.bfloat16)
    w3b = w3_ref[0].astype(jnp.bfloat16)
    a = jax.lax.dot_general(xb, w1b, (((1,), (1,)), ((), ())),
                            preferred_element_type=jnp.float32)
    g = jax.lax.dot_general(xb, w3b, (((1,), (1,)), ((), ())),
                            preferred_element_type=jnp.float32)
    h = (a * jax.nn.sigmoid(a)) * g
    w2b = w2_ref[0].astype(jnp.bfloat16)
    contrib = jax.lax.dot_general(h.astype(jnp.bfloat16), w2b,
                                  (((1,), (1,)), ((), ())),
                                  preferred_element_type=jnp.float32)
    tokens = contrib.shape[0]
    eidx = jax.lax.broadcasted_iota(jnp.int32, (tokens, _NUM_EXPERTS), 1)
    wcol = jnp.sum(jnp.where(eidx == e, wts_ref[...], 0.0), axis=1,
                   keepdims=True)
    upd = contrib * wcol

    @pl.when(first)
    def _():
        out_ref[...] = upd

    @pl.when(jnp.logical_not(first))
    def _():
        out_ref[...] += upd


def kernel(hidden_states, gate_w, w1, w2, w3):
    b, s, hdim = hidden_states.shape
    tokens = b * s
    x = hidden_states.reshape(tokens, hdim)
    x_r = lax.optimization_barrier(x.astype(jnp.bfloat16)).astype(jnp.float32)
    gw_r = lax.optimization_barrier(gate_w.astype(jnp.bfloat16)).astype(jnp.float32)
    wts = _sc_router(x_r, gw_r)
    out = pl.pallas_call(
        _moe_body,
        grid=(_NUM_EXPERTS, _NC),
        in_specs=[
            pl.BlockSpec((tokens, _HIDDEN), lambda e, c: (0, 0)),
            pl.BlockSpec((tokens, _NUM_EXPERTS), lambda e, c: (0, 0)),
            pl.BlockSpec((1, _IC, _HIDDEN), lambda e, c: (e, c, 0)),
            pl.BlockSpec((1, _IC, _HIDDEN), lambda e, c: (e, c, 0)),
            pl.BlockSpec((1, _HIDDEN, _IC), lambda e, c: (e, 0, c)),
        ],
        out_specs=pl.BlockSpec((tokens, _HIDDEN), lambda e, c: (0, 0)),
        out_shape=jax.ShapeDtypeStruct((tokens, _HIDDEN), jnp.float32),
        compiler_params=pltpu.CompilerParams(
            dimension_semantics=("arbitrary", "arbitrary")),
    )(x, wts, w1, w3, w2)
    return out.reshape(b, s, hdim)


# speedup vs baseline: 1.0953x; 1.0369x over previous
import jax
import jax.numpy as jnp
from jax import lax
from jax.experimental import pallas as pl
from jax.experimental.pallas import tpu as pltpu
from jax.experimental.pallas import tpu_sc as plsc

_NUM_EXPERTS = 16
_HIDDEN = 1024
_INTER = 2048
_JITTER = 0.01
_IC = 1024
_NC = _INTER // _IC
_LANES = 16
_NWORKERS = 32


def _sparsemixer_row(lv):
    neg_inf = jnp.float32(-jnp.inf)
    lane = lax.iota(jnp.int32, _LANES)
    t1 = lax.reduce_max(lv, (0,))
    oh1 = lane == plsc.all_reduce_ffs(lv == t1)
    factor1 = jnp.maximum(jnp.abs(lv), t1)
    mask1 = ((t1 - lv) / factor1) > (2.0 * _JITTER)
    p1 = jnp.exp(jnp.where(mask1, neg_inf, lv) - t1)
    den1 = jnp.broadcast_to(lax.reduce_sum(p1, (0,)), (_LANES,))
    row1 = jnp.where(oh1, p1, 0.0) / den1
    s2 = jnp.where(oh1, neg_inf, lv)
    t2 = lax.reduce_max(s2, (0,))
    oh2 = lane == plsc.all_reduce_ffs(s2 == t2)
    factor2 = jnp.maximum(jnp.abs(lv), t2)
    mask2 = ((t2 - s2) / factor2) > (2.0 * _JITTER)
    p2 = jnp.exp(jnp.where(mask2, neg_inf, s2) - t2)
    den2 = jnp.broadcast_to(lax.reduce_sum(p2, (0,)), (_LANES,))
    row2 = jnp.where(oh2, p2, 0.0) / den2
    return row1 + row2


def _router_body(lg_hbm, out_hbm, lgv, wv):
    wid = lax.axis_index("s") * 2 + lax.axis_index("c")
    tpw = 4
    base = wid * tpw
    pltpu.sync_copy(lg_hbm.at[pl.ds(base, tpw)], lgv)
    for t in range(tpw):
        wv[t, :] = _sparsemixer_row(lgv[t, :])
    pltpu.sync_copy(wv, out_hbm.at[pl.ds(base, tpw)])


def _sc_router(logits):
    tokens = logits.shape[0]
    mesh = plsc.VectorSubcoreMesh(core_axis_name="c", subcore_axis_name="s")
    tpw = tokens // _NWORKERS
    run = pl.kernel(
        _router_body, mesh=mesh,
        out_type=jax.ShapeDtypeStruct((tokens, _NUM_EXPERTS), jnp.float32),
        scratch_types=[
            pltpu.VMEM((tpw, _NUM_EXPERTS), jnp.float32),
            pltpu.VMEM((tpw, _NUM_EXPERTS), jnp.float32),
        ],
        compiler_params=pltpu.CompilerParams(needs_layout_passes=False),
    )
    return run(logits)


def _logits_body(x_ref, gw_ref, out_ref):
    out_ref[...] = jax.lax.dot_general(
        x_ref[...].astype(jnp.bfloat16), gw_ref[...].astype(jnp.bfloat16),
        (((1,), (1,)), ((), ())), preferred_element_type=jnp.float32)


def _moe_body(x_ref, wts_ref, w1_ref, w3_ref, w2_ref, out_ref):
    e = pl.program_id(0)
    c = pl.program_id(1)
    first = (e == 0) & (c == 0)

    xb = x_ref[...].astype(jnp.bfloat16)
    w1b = w1_ref[0].astype(jnp.bfloat16)
    w3b = w3_ref[0].astype(jnp.bfloat16)
    a = jax.lax.dot_general(xb, w1b, (((1,), (1,)), ((), ())),
                            preferred_element_type=jnp.float32)
    g = jax.lax.dot_general(xb, w3b, (((1,), (1,)), ((), ())),
                            preferred_element_type=jnp.float32)
    h = (a * jax.nn.sigmoid(a)) * g
    w2b = w2_ref[0].astype(jnp.bfloat16)
    contrib = jax.lax.dot_general(h.astype(jnp.bfloat16), w2b,
                                  (((1,), (1,)), ((), ())),
                                  preferred_element_type=jnp.float32)
    tokens = contrib.shape[0]
    eidx = jax.lax.broadcasted_iota(jnp.int32, (tokens, _NUM_EXPERTS), 1)
    wcol = jnp.sum(jnp.where(eidx == e, wts_ref[...], 0.0), axis=1,
                   keepdims=True)
    upd = contrib * wcol

    @pl.when(first)
    def _():
        out_ref[...] = upd

    @pl.when(jnp.logical_not(first))
    def _():
        out_ref[...] += upd


def kernel(hidden_states, gate_w, w1, w2, w3):
    b, s, hdim = hidden_states.shape
    tokens = b * s
    x = hidden_states.reshape(tokens, hdim)
    logits = pl.pallas_call(
        _logits_body,
        out_shape=jax.ShapeDtypeStruct((tokens, _NUM_EXPERTS), jnp.float32),
    )(x, gate_w)
    wts = _sc_router(logits)
    out = pl.pallas_call(
        _moe_body,
        grid=(_NUM_EXPERTS, _NC),
        in_specs=[
            pl.BlockSpec((tokens, _HIDDEN), lambda e, c: (0, 0)),
            pl.BlockSpec((tokens, _NUM_EXPERTS), lambda e, c: (0, 0)),
            pl.BlockSpec((1, _IC, _HIDDEN), lambda e, c: (e, c, 0)),
            pl.BlockSpec((1, _IC, _HIDDEN), lambda e, c: (e, c, 0)),
            pl.BlockSpec((1, _HIDDEN, _IC), lambda e, c: (e, 0, c)),
        ],
        out_specs=pl.BlockSpec((tokens, _HIDDEN), lambda e, c: (0, 0)),
        out_shape=jax.ShapeDtypeStruct((tokens, _HIDDEN), jnp.float32),
        compiler_params=pltpu.CompilerParams(
            dimension_semantics=("arbitrary", "arbitrary")),
    )(x, wts, w1, w3, w2)
    return out.reshape(b, s, hdim)
